# baseline (speedup 1.0000x reference)
import jax
import jax.numpy as jnp
from jax.experimental import pallas as pl

BATCH = 16
MAX_LAB = 14
NROW = 4
IMG_W = 640.0
IMG_H = 640.0
N_DET = 20000
N_GT = BATCH * MAX_LAB
CHUNK = 2000
N_CHUNKS = N_DET // CHUNK
BIG = 3.0e9


def _tc_body(pp_ref, imgrow_ref, praw_ref, cls_ref, out_ref, aid_ref):
    imgrow = imgrow_ref[...]
    g = jax.lax.broadcasted_iota(jnp.int32, (1, N_GT), 1)
    lab = g % MAX_LAB
    slot = g // MAX_LAB
    oh2 = (jax.lax.broadcasted_iota(jnp.int32, (BATCH, N_GT), 0)
           == (imgrow % BATCH)).astype(jnp.float32)
    labm = (jax.lax.broadcasted_iota(jnp.int32, (MAX_LAB, N_GT), 0)
            == lab).astype(jnp.float32)

    def pickrow(c):
        sel = jnp.dot(pp_ref[c * MAX_LAB:(c + 1) * MAX_LAB, :], oh2,
                      preferred_element_type=jnp.float32)
        return jnp.sum(sel * labm, axis=0, keepdims=True)

    cxs = pickrow(0)
    cys = pickrow(1)
    ws = pickrow(2)
    hs = pickrow(3)

    offx = IMG_W * (slot % NROW).astype(jnp.float32)
    offy = IMG_H * (slot // NROW).astype(jnp.float32)
    nz = (cxs + cys + ws + hs) != 0.0
    cxp = cxs * IMG_W + jnp.where(nz, offx, 0.0)
    cyp = cys * IMG_H + jnp.where(nz, offy, 0.0)
    bw = IMG_W * ws
    bh = IMG_H * hs
    x1 = cxp - bw * 0.5
    y1 = cyp - bh * 0.5
    x2 = x1 + bw
    y2 = y1 + bh
    needed = (x1 + y1 + x2 + y2) != 0.0
    gx1 = jnp.where(needed, x1, 0.0)
    gy1 = jnp.where(needed, y1, 0.0)
    gx2 = jnp.where(needed, x2, 0.0)
    gy2 = jnp.where(needed, y2, 0.0)
    aidr = jnp.where(needed & (imgrow >= BATCH), 1, 0).astype(jnp.int32)
    gag = (gx2 - gx1 + 1.0) * (gy2 - gy1 + 1.0)
    gx2p = gx2 + 1.0
    gy2p = gy2 + 1.0

    mall = cls_ref[...] == 0
    anyb = jnp.max(mall.astype(jnp.float32)) > 0.0

    ones_gt = jnp.ones((1, N_GT), jnp.float32)

    def chunk_body(c, acc):
        ss = pl.ds(c * CHUNK, CHUNK)
        blk = praw_ref[ss, :]
        mc = cls_ref[ss, :] == 0
        x1m = jnp.where(mc, blk[:, 0:1], BIG)
        y1c = blk[:, 1:2]
        x2p = blk[:, 2:3] + 1.0
        y2p = blk[:, 3:4] + 1.0
        areab = (x2p - x1m) * (y2p - y1c)

        def bc(col):
            return jnp.dot(col, ones_gt, preferred_element_type=jnp.float32)

        bx1 = bc(x1m)
        by1 = bc(y1c)
        bx2 = bc(x2p)
        by2 = bc(y2p)
        bab = bc(areab)
        iw = jnp.maximum(jnp.minimum(gx2p, bx2) - jnp.maximum(gx1, bx1), 0.0)
        ih = jnp.maximum(jnp.minimum(gy2p, by2) - jnp.maximum(gy1, by1), 0.0)
        inters = iw * ih
        uni = gag + bab - inters
        return jnp.maximum(acc, jnp.max(inters / uni, axis=0, keepdims=True))

    acc0 = jnp.zeros((1, N_GT), jnp.float32)
    ov = jax.lax.fori_loop(0, N_CHUNKS, chunk_body, acc0)

    iou_pred = jnp.concatenate([ov, 1.0 - ov], axis=0)
    aidf = aidr.astype(jnp.float32)
    basep = jnp.concatenate([aidf, jnp.abs(aidf - 1.0)], axis=0)
    out_ref[...] = jnp.where(anyb, iou_pred, basep) * 10.0
    aid_ref[...] = aidr


def kernel(people_boxes, pred_boxes, pred_scores, pred_classes, image_index):
    del pred_scores
    pp = jnp.transpose(people_boxes, (2, 1, 0)).reshape(4 * MAX_LAB, BATCH)
    imgrow = jnp.repeat(image_index, MAX_LAB).reshape(1, N_GT)
    clscol = pred_classes.reshape(N_DET, 1)

    predt, aidrow = pl.pallas_call(
        _tc_body,
        out_shape=[
            jax.ShapeDtypeStruct((2, N_GT), jnp.float32),
            jax.ShapeDtypeStruct((1, N_GT), jnp.int32),
        ],
    )(pp, imgrow, pred_boxes, clscol)
    return (predt.T, aidrow.reshape(N_GT))


# speedup vs baseline: 2.7838x; 2.7838x over previous
import jax
import jax.numpy as jnp
from jax.experimental import pallas as pl
from jax.experimental.pallas import tpu as pltpu

BATCH = 16
MAX_LAB = 14
NROW = 4
IMG_W = 640.0
IMG_H = 640.0
N_DET = 20000
N_PAD = 20480
N_GT = BATCH * MAX_LAB
CHUNK = 2048
N_CHUNKS = N_PAD // CHUNK


def _tc_body(people_ref, imgrep_ref, pred_ref, out_ref, aid_ref, mod_ref):
    imgrep = imgrep_ref[...]
    giota = jax.lax.broadcasted_iota(jnp.int32, (N_GT, 1), 0)
    lab = giota % MAX_LAB
    ohT = (jax.lax.broadcasted_iota(jnp.int32, (N_GT, BATCH), 1)
           == (imgrep % BATCH)).astype(jnp.float32)
    colmask = (jax.lax.broadcasted_iota(jnp.int32, (N_GT, MAX_LAB), 1)
               == lab).astype(jnp.float32)

    def pick(rows):
        sel = jnp.dot(ohT, rows, preferred_element_type=jnp.float32)
        return jnp.sum(sel * colmask, axis=1, keepdims=True)

    cxs = pick(people_ref[0:16, :])
    cys = pick(people_ref[16:32, :])
    ws = pick(people_ref[32:48, :])
    hs = pick(people_ref[48:64, :])

    slot = giota // MAX_LAB
    offx = IMG_W * (slot % NROW).astype(jnp.float32)
    offy = IMG_H * (slot // NROW).astype(jnp.float32)
    nz = (cxs + cys + ws + hs) != 0.0
    cxp = cxs * IMG_W + jnp.where(nz, offx, 0.0)
    cyp = cys * IMG_H + jnp.where(nz, offy, 0.0)
    bw = IMG_W * ws
    bh = IMG_H * hs
    x1 = cxp - bw * 0.5
    y1 = cyp - bh * 0.5
    x2 = x1 + bw
    y2 = y1 + bh
    needed = (x1 + y1 + x2 + y2) != 0.0
    c_x1 = jnp.where(needed, x1, 0.0)
    c_y1 = jnp.where(needed, y1, 0.0)
    c_x2 = jnp.where(needed, x2, 0.0)
    c_y2 = jnp.where(needed, y2, 0.0)
    aidc = jnp.where(needed & (imgrep >= BATCH), 1, 0).astype(jnp.int32)
    c_ag = (c_x2 - c_x1 + 1.0) * (c_y2 - c_y1 + 1.0)

    clsrow = pred_ref[4:5, :]
    m = clsrow == 0.0
    anyb = jnp.max(m.astype(jnp.float32)) > 0.0
    mod_ref[0:1, :] = jnp.where(m, pred_ref[0:1, :], 3.0e9)
    mod_ref[1:2, :] = pred_ref[2:3, :] + 1.0
    mod_ref[2:3, :] = pred_ref[3:4, :] + 1.0
    zrow = jnp.zeros((1, CHUNK), jnp.float32)
    gx1b = c_x1 + zrow
    gy1b = c_y1 + zrow
    gx2b = (c_x2 + 1.0) + zrow
    gy2b = (c_y2 + 1.0) + zrow
    gagb = c_ag + zrow

    def chunk_body(c, acc):
        sl = pl.ds(c * CHUNK, CHUNK)
        px1 = mod_ref[0:1, sl]
        py1 = pred_ref[1:2, sl]
        px2p = mod_ref[1:2, sl]
        py2p = mod_ref[2:3, sl]
        areab = (px2p - pred_ref[0:1, sl]) * (py2p - py1)
        iw = jnp.maximum(jnp.minimum(gx2b, px2p) - jnp.maximum(gx1b, px1), 0.0)
        ih = jnp.maximum(jnp.minimum(gy2b, py2p) - jnp.maximum(gy1b, py1), 0.0)
        inters = iw * ih
        uni = gagb + areab - inters
        return jnp.maximum(acc, jnp.max(inters / uni, axis=1, keepdims=True))

    ov = jnp.zeros((N_GT, 1), jnp.float32)
    for c in range(N_CHUNKS):
        ov = chunk_body(c, ov)

    iou_pred = jnp.concatenate([ov, 1.0 - ov], axis=1)
    aidf = aidc.astype(jnp.float32)
    basep = jnp.concatenate([aidf, jnp.abs(aidf - 1.0)], axis=1)
    out_ref[...] = jnp.where(anyb, iou_pred, basep) * 10.0
    aid_ref[...] = aidc


def kernel(people_boxes, pred_boxes, pred_scores, pred_classes, image_index):
    del pred_scores
    people = jnp.transpose(people_boxes, (2, 0, 1)).reshape(64, MAX_LAB)
    imgrep = jnp.repeat(image_index, MAX_LAB).reshape(N_GT, 1)
    coords = pred_boxes.T
    clsf = pred_classes.astype(jnp.float32)[None, :]
    main = jnp.concatenate([coords, clsf], axis=0)
    padblk = jnp.zeros((5, N_PAD - N_DET), jnp.float32).at[4].set(1.0)
    predT = jnp.concatenate([main, padblk], axis=1)

    predict, aid = pl.pallas_call(
        _tc_body,
        out_shape=[
            jax.ShapeDtypeStruct((N_GT, 2), jnp.float32),
            jax.ShapeDtypeStruct((N_GT, 1), jnp.int32),
        ],
        scratch_shapes=[pltpu.VMEM((3, N_PAD), jnp.float32)],
    )(people, imgrep, predT)
    return (predict, aid.reshape(N_GT))


# speedup vs baseline: 2.8085x; 1.0089x over previous
import jax
import jax.numpy as jnp
from jax.experimental import pallas as pl
from jax.experimental.pallas import tpu as pltpu

BATCH = 16
MAX_LAB = 14
NROW = 4
IMG_W = 640.0
IMG_H = 640.0
N_DET = 20000
N_GT = BATCH * MAX_LAB
CHUNK = 2048
TAIL = N_DET - 9 * CHUNK


def _tc_body(people_ref, imgrep_ref, pred_ref, out_ref, aid_ref, mod_ref):
    imgrep = imgrep_ref[...]
    giota = jax.lax.broadcasted_iota(jnp.int32, (N_GT, 1), 0)
    lab = giota % MAX_LAB
    ohT = (jax.lax.broadcasted_iota(jnp.int32, (N_GT, BATCH), 1)
           == (imgrep % BATCH)).astype(jnp.float32)
    colmask = (jax.lax.broadcasted_iota(jnp.int32, (N_GT, MAX_LAB), 1)
               == lab).astype(jnp.float32)

    def pick(rows):
        sel = jnp.dot(ohT, rows, preferred_element_type=jnp.float32)
        return jnp.sum(sel * colmask, axis=1, keepdims=True)

    cxs = pick(people_ref[0:16, :])
    cys = pick(people_ref[16:32, :])
    ws = pick(people_ref[32:48, :])
    hs = pick(people_ref[48:64, :])

    slot = giota // MAX_LAB
    offx = IMG_W * (slot % NROW).astype(jnp.float32)
    offy = IMG_H * (slot // NROW).astype(jnp.float32)
    nz = (cxs + cys + ws + hs) != 0.0
    cxp = cxs * IMG_W + jnp.where(nz, offx, 0.0)
    cyp = cys * IMG_H + jnp.where(nz, offy, 0.0)
    bw = IMG_W * ws
    bh = IMG_H * hs
    x1 = cxp - bw * 0.5
    y1 = cyp - bh * 0.5
    x2 = x1 + bw
    y2 = y1 + bh
    needed = (x1 + y1 + x2 + y2) != 0.0
    c_x1 = jnp.where(needed, x1, 0.0)
    c_y1 = jnp.where(needed, y1, 0.0)
    c_x2 = jnp.where(needed, x2, 0.0)
    c_y2 = jnp.where(needed, y2, 0.0)
    aidc = jnp.where(needed & (imgrep >= BATCH), 1, 0).astype(jnp.int32)
    c_ag = (c_x2 - c_x1 + 1.0) * (c_y2 - c_y1 + 1.0)

    clsrow = pred_ref[4:5, :]
    m = clsrow == 0.0
    anyb = jnp.max(m.astype(jnp.float32)) > 0.0
    mod_ref[0:1, :] = jnp.where(m, pred_ref[0:1, :], 3.0e9)
    mod_ref[1:2, :] = pred_ref[2:3, :] + 1.0
    mod_ref[2:3, :] = pred_ref[3:4, :] + 1.0
    zrow = jnp.zeros((1, CHUNK), jnp.float32)
    gx1b = c_x1 + zrow
    gy1b = c_y1 + zrow
    gx2b = (c_x2 + 1.0) + zrow
    gy2b = (c_y2 + 1.0) + zrow
    gagb = c_ag + zrow

    def chunk_body(start, width, acc):
        sl = pl.ds(start, width)
        px1 = mod_ref[0:1, sl]
        py1 = pred_ref[1:2, sl]
        px2p = mod_ref[1:2, sl]
        py2p = mod_ref[2:3, sl]
        areab = (px2p - pred_ref[0:1, sl]) * (py2p - py1)
        iw = jnp.maximum(jnp.minimum(gx2b[:, :width], px2p)
                         - jnp.maximum(gx1b[:, :width], px1), 0.0)
        ih = jnp.maximum(jnp.minimum(gy2b[:, :width], py2p)
                         - jnp.maximum(gy1b[:, :width], py1), 0.0)
        inters = iw * ih
        uni = gagb[:, :width] + areab - inters
        return jnp.maximum(acc, jnp.max(inters / uni, axis=1, keepdims=True))

    ov = jnp.zeros((N_GT, 1), jnp.float32)
    for c in range(9):
        ov = chunk_body(c * CHUNK, CHUNK, ov)
    ov = chunk_body(9 * CHUNK, TAIL, ov)

    iou_pred = jnp.concatenate([ov, 1.0 - ov], axis=1)
    aidf = aidc.astype(jnp.float32)
    basep = jnp.concatenate([aidf, jnp.abs(aidf - 1.0)], axis=1)
    out_ref[...] = jnp.where(anyb, iou_pred, basep) * 10.0
    aid_ref[...] = aidc


def kernel(people_boxes, pred_boxes, pred_scores, pred_classes, image_index):
    del pred_scores
    people = jnp.transpose(people_boxes, (2, 0, 1)).reshape(64, MAX_LAB)
    imgrep = jnp.repeat(image_index, MAX_LAB).reshape(N_GT, 1)
    coords = pred_boxes.T
    clsf = pred_classes.astype(jnp.float32)[None, :]
    predT = jnp.concatenate([coords, clsf], axis=0)

    predict, aid = pl.pallas_call(
        _tc_body,
        out_shape=[
            jax.ShapeDtypeStruct((N_GT, 2), jnp.float32),
            jax.ShapeDtypeStruct((N_GT, 1), jnp.int32),
        ],
        scratch_shapes=[pltpu.VMEM((3, N_DET), jnp.float32)],
    )(people, imgrep, predT)
    return (predict, aid.reshape(N_GT))


# speedup vs baseline: 2.8934x; 1.0303x over previous
import jax
import jax.numpy as jnp
from jax.experimental import pallas as pl
from jax.experimental.pallas import tpu as pltpu

BATCH = 16
MAX_LAB = 14
NROW = 4
IMG_W = 640.0
IMG_H = 640.0
N_DET = 20000
N_GT = BATCH * MAX_LAB
CHUNK = 2048
TAIL = N_DET - 9 * CHUNK


def _tc_body(people_ref, imgrep_ref, pred_ref, cls_ref, out_ref, aid_ref, mod_ref):
    imgrep = imgrep_ref[...]
    giota = jax.lax.broadcasted_iota(jnp.int32, (N_GT, 1), 0)
    lab = giota % MAX_LAB
    ohT = (jax.lax.broadcasted_iota(jnp.int32, (N_GT, BATCH), 1)
           == (imgrep % BATCH)).astype(jnp.float32)
    colmask = (jax.lax.broadcasted_iota(jnp.int32, (N_GT, MAX_LAB), 1)
               == lab).astype(jnp.float32)

    def pick(rows):
        sel = jnp.dot(ohT, rows, preferred_element_type=jnp.float32)
        return jnp.sum(sel * colmask, axis=1, keepdims=True)

    cxs = pick(people_ref[0:16, :])
    cys = pick(people_ref[16:32, :])
    ws = pick(people_ref[32:48, :])
    hs = pick(people_ref[48:64, :])

    slot = giota // MAX_LAB
    offx = IMG_W * (slot % NROW).astype(jnp.float32)
    offy = IMG_H * (slot // NROW).astype(jnp.float32)
    nz = (cxs + cys + ws + hs) != 0.0
    cxp = cxs * IMG_W + jnp.where(nz, offx, 0.0)
    cyp = cys * IMG_H + jnp.where(nz, offy, 0.0)
    bw = IMG_W * ws
    bh = IMG_H * hs
    x1 = cxp - bw * 0.5
    y1 = cyp - bh * 0.5
    x2 = x1 + bw
    y2 = y1 + bh
    needed = (x1 + y1 + x2 + y2) != 0.0
    c_x1 = jnp.where(needed, x1, 0.0)
    c_y1 = jnp.where(needed, y1, 0.0)
    c_x2 = jnp.where(needed, x2, 0.0)
    c_y2 = jnp.where(needed, y2, 0.0)
    aidc = jnp.where(needed & (imgrep >= BATCH), 1, 0).astype(jnp.int32)
    c_ag = (c_x2 - c_x1 + 1.0) * (c_y2 - c_y1 + 1.0)

    m = cls_ref[...] == 0.0
    anyb = jnp.max(m.astype(jnp.float32)) > 0.0
    mod_ref[0:1, :] = jnp.where(m, pred_ref[0:1, :], 3.0e9)
    mod_ref[1:2, :] = pred_ref[2:3, :] + 1.0
    mod_ref[2:3, :] = pred_ref[3:4, :] + 1.0
    zrow = jnp.zeros((1, CHUNK), jnp.float32)
    gx1b = c_x1 + zrow
    gy1b = c_y1 + zrow
    gx2b = (c_x2 + 1.0) + zrow
    gy2b = (c_y2 + 1.0) + zrow
    gagb = c_ag + zrow

    def chunk_body(start, width, acc):
        sl = pl.ds(start, width)
        px1 = mod_ref[0:1, sl]
        py1 = pred_ref[1:2, sl]
        px2p = mod_ref[1:2, sl]
        py2p = mod_ref[2:3, sl]
        areab = (px2p - pred_ref[0:1, sl]) * (py2p - py1)
        iw = jnp.maximum(jnp.minimum(gx2b[:, :width], px2p)
                         - jnp.maximum(gx1b[:, :width], px1), 0.0)
        ih = jnp.maximum(jnp.minimum(gy2b[:, :width], py2p)
                         - jnp.maximum(gy1b[:, :width], py1), 0.0)
        inters = iw * ih
        uni = gagb[:, :width] + areab - inters
        return jnp.maximum(acc, jnp.max(inters / uni, axis=1, keepdims=True))

    ov = jnp.zeros((N_GT, 1), jnp.float32)
    for c in range(9):
        ov = chunk_body(c * CHUNK, CHUNK, ov)
    ov = chunk_body(9 * CHUNK, TAIL, ov)

    iou_pred = jnp.concatenate([ov, 1.0 - ov], axis=1)
    aidf = aidc.astype(jnp.float32)
    basep = jnp.concatenate([aidf, jnp.abs(aidf - 1.0)], axis=1)
    out_ref[...] = jnp.where(anyb, iou_pred, basep) * 10.0
    aid_ref[...] = aidc


def kernel(people_boxes, pred_boxes, pred_scores, pred_classes, image_index):
    del pred_scores
    people = jnp.transpose(people_boxes, (2, 0, 1)).reshape(64, MAX_LAB)
    imgrep = jnp.repeat(image_index, MAX_LAB).reshape(N_GT, 1)
    coords = pred_boxes.T
    clsrow = pred_classes.astype(jnp.float32).reshape(1, N_DET)

    predict, aid = pl.pallas_call(
        _tc_body,
        out_shape=[
            jax.ShapeDtypeStruct((N_GT, 2), jnp.float32),
            jax.ShapeDtypeStruct((N_GT, 1), jnp.int32),
        ],
        scratch_shapes=[pltpu.VMEM((3, N_DET), jnp.float32)],
    )(people, imgrep, coords, clsrow)
    return (predict, aid.reshape(N_GT))
